# Initial kernel scaffold; baseline (speedup 1.0000x reference)
import jax
import jax.numpy as jnp
from jax.experimental import pallas as pl


def kernel(x, table, W1, b1, W2, b2):
    raise NotImplementedError("write your pallas kernel here")


# speedup vs baseline: 1.2675x; 1.2675x over previous
import functools

import jax
import jax.numpy as jnp
from jax import lax
from jax.experimental import pallas as pl
from jax.experimental.pallas import tpu as pltpu
from jax.experimental.pallas import tpu_sc as plsc

L = 16


@functools.cache
def _make_bag_kernel(B, H, D):
    info = plsc.get_sparse_core_info()
    NC, NS = info.num_cores, info.num_subcores
    NW = NC * NS
    nb = B // NW
    HH = H // 2
    nd = D // L

    mesh = plsc.VectorSubcoreMesh(core_axis_name="c", subcore_axis_name="s")

    @functools.partial(
        pl.kernel,
        mesh=mesh,
        out_type=jax.ShapeDtypeStruct((B, D), jnp.float32),
        scratch_types=[
            pltpu.VMEM((nb, 2, HH), jnp.int32),
            pltpu.VMEM((H, D), jnp.float32),
            pltpu.VMEM((nb, D), jnp.float32),
            pltpu.SemaphoreType.DMA,
        ],
    )
    def bag_kernel(x_hbm, table_hbm, out_hbm, idx_v, rows_v, pooled_v, sem):
        wid = lax.axis_index("s") * NC + lax.axis_index("c")
        base = wid * nb
        pltpu.sync_copy(x_hbm.at[pl.ds(base, nb)], idx_v)

        def bag_body(i, carry):
            cp0 = pltpu.async_copy(
                table_hbm.at[idx_v.at[i, 0]], rows_v.at[pl.ds(0, HH)], sem)
            cp1 = pltpu.async_copy(
                table_hbm.at[idx_v.at[i, 1]], rows_v.at[pl.ds(HH, HH)], sem)
            cp0.wait()
            cp1.wait()

            def row_body(r, accs):
                return tuple(
                    accs[d] + rows_v[r, pl.ds(d * L, L)] for d in range(nd))

            accs = tuple(jnp.zeros((L,), jnp.float32) for _ in range(nd))
            accs = lax.fori_loop(0, H, row_body, accs)
            for d in range(nd):
                pooled_v[i, pl.ds(d * L, L)] = accs[d]
            return carry

        lax.fori_loop(0, nb, bag_body, 0)
        pltpu.sync_copy(pooled_v, out_hbm.at[pl.ds(base, nb)])

    return bag_kernel


def _mlp_body(p_ref, w1_ref, b1_ref, w2_ref, b2_ref, o_ref):
    h = lax.dot_general(p_ref[...], w1_ref[...], (((1,), (1,)), ((), ())),
                        preferred_element_type=jnp.float32)
    h = jnp.maximum(h + b1_ref[...], 0.0)
    o_ref[...] = lax.dot_general(h, w2_ref[...], (((1,), (1,)), ((), ())),
                                 preferred_element_type=jnp.float32) + b2_ref[...]


def kernel(x, table, W1, b1, W2, b2):
    B, H = x.shape
    D = table.shape[1]
    x3 = x.reshape(B, 2, H // 2).astype(jnp.int32)
    pooled = _make_bag_kernel(B, H, D)(x3, table)
    out = pl.pallas_call(
        _mlp_body,
        out_shape=jax.ShapeDtypeStruct((B, W2.shape[0]), jnp.float32),
    )(pooled, W1, b1.reshape(1, -1), W2, b2.reshape(1, -1))
    return out


# speedup vs baseline: 2.7263x; 2.1509x over previous
import functools

import jax
import jax.numpy as jnp
from jax import lax
from jax.experimental import pallas as pl
from jax.experimental.pallas import tpu as pltpu
from jax.experimental.pallas import tpu_sc as plsc

L = 16


@functools.cache
def _make_bag_kernel(B, H, D):
    info = plsc.get_sparse_core_info()
    NC, NS = info.num_cores, info.num_subcores
    NW = NC * NS
    nb = B // NW
    HH = H // 2
    nd = D // L

    mesh = plsc.VectorSubcoreMesh(core_axis_name="c", subcore_axis_name="s")

    @functools.partial(
        pl.kernel,
        mesh=mesh,
        out_type=jax.ShapeDtypeStruct((B, D), jnp.float32),
        scratch_types=[
            pltpu.VMEM((nb, 2, HH), jnp.int32),
            pltpu.VMEM((4, HH, D), jnp.float32),
            pltpu.VMEM((nb, D), jnp.float32),
            pltpu.SemaphoreType.DMA,
            pltpu.SemaphoreType.DMA,
            pltpu.SemaphoreType.DMA,
            pltpu.SemaphoreType.DMA,
        ],
    )
    def bag_kernel(x_hbm, table_hbm, out_hbm, idx_v, rows_v, pooled_v,
                   s0, s1, s2, s3):
        sems = (s0, s1, s2, s3)
        wid = lax.axis_index("s") * NC + lax.axis_index("c")
        base = wid * nb
        pltpu.sync_copy(x_hbm.at[pl.ds(base, nb)], idx_v)

        def gather(bag, hb, b):
            return pltpu.make_async_copy(
                table_hbm.at[idx_v.at[bag, hb]], rows_v.at[b], sems[b])

        for b in range(4):
            gather(b // 2, b % 2, b).start()

        def body(i, carry):
            for j in range(2):
                bag = 2 * i + j
                accs = tuple(jnp.zeros((L,), jnp.float32) for _ in range(nd))
                for hb in range(2):
                    b = 2 * j + hb
                    gather(bag, hb, b).wait()

                    def row_body(r, a, _b=b):
                        return tuple(
                            a[d] + rows_v[_b, r, pl.ds(d * L, L)]
                            for d in range(nd))

                    accs = lax.fori_loop(0, HH, row_body, accs)

                    @pl.when(bag + 2 < nb)
                    def _():
                        gather(bag + 2, hb, b).start()

                for d in range(nd):
                    pooled_v[bag, pl.ds(d * L, L)] = accs[d]
            return carry

        lax.fori_loop(0, nb // 2, body, 0)
        pltpu.sync_copy(pooled_v, out_hbm.at[pl.ds(base, nb)])

    return bag_kernel


def _mlp_body(p_ref, w1_ref, b1_ref, w2_ref, b2_ref, o_ref):
    h = lax.dot_general(p_ref[...], w1_ref[...], (((1,), (1,)), ((), ())),
                        preferred_element_type=jnp.float32)
    h = jnp.maximum(h + b1_ref[...], 0.0)
    o_ref[...] = lax.dot_general(h, w2_ref[...], (((1,), (1,)), ((), ())),
                                 preferred_element_type=jnp.float32) + b2_ref[...]


def kernel(x, table, W1, b1, W2, b2):
    B, H = x.shape
    D = table.shape[1]
    x3 = x.reshape(B, 2, H // 2).astype(jnp.int32)
    pooled = _make_bag_kernel(B, H, D)(x3, table)
    out = pl.pallas_call(
        _mlp_body,
        out_shape=jax.ShapeDtypeStruct((B, W2.shape[0]), jnp.float32),
    )(pooled, W1, b1.reshape(1, -1), W2, b2.reshape(1, -1))
    return out


# speedup vs baseline: 2.7310x; 1.0017x over previous
import functools

import jax
import jax.numpy as jnp
from jax import lax
from jax.experimental import pallas as pl
from jax.experimental.pallas import tpu as pltpu
from jax.experimental.pallas import tpu_sc as plsc

L = 16


@functools.cache
def _make_bag_kernel(B, H, D):
    info = plsc.get_sparse_core_info()
    NC, NS = info.num_cores, info.num_subcores
    NW = NC * NS
    nb = B // NW
    HH = H // 2
    nd = D // L

    mesh = plsc.VectorSubcoreMesh(core_axis_name="c", subcore_axis_name="s")

    @functools.partial(
        pl.kernel,
        mesh=mesh,
        out_type=jax.ShapeDtypeStruct((B, D), jnp.float32),
        scratch_types=[
            pltpu.VMEM((nb, 2, HH), jnp.int32),
            pltpu.VMEM((4, HH, D), jnp.float32),
            pltpu.VMEM((nb, D), jnp.float32),
            pltpu.SemaphoreType.DMA,
            pltpu.SemaphoreType.DMA,
            pltpu.SemaphoreType.DMA,
            pltpu.SemaphoreType.DMA,
        ],
    )
    def bag_kernel(x_hbm, table_hbm, out_hbm, idx_v, rows_v, pooled_v,
                   s0, s1, s2, s3):
        sems = (s0, s1, s2, s3)
        wid = lax.axis_index("s") * NC + lax.axis_index("c")
        base = wid * nb
        pltpu.sync_copy(x_hbm.at[pl.ds(base, nb)], idx_v)

        def gather(bag, hb, b):
            return pltpu.make_async_copy(
                table_hbm.at[idx_v.at[bag, hb]], rows_v.at[b], sems[b])

        for b in range(4):
            gather(b // 2, b % 2, b).start()

        def body(i, carry):
            for j in range(2):
                bag = 2 * i + j
                accs = tuple(jnp.zeros((L,), jnp.float32) for _ in range(nd))
                for hb in range(2):
                    b = 2 * j + hb
                    gather(bag, hb, b).wait()

                    def row_body(r, a, _b=b):
                        r2 = 2 * r
                        for rr in (r2, r2 + 1):
                            a = tuple(
                                a[d] + rows_v[_b, rr, pl.ds(d * L, L)]
                                for d in range(nd))
                        return a

                    accs = lax.fori_loop(0, HH // 2, row_body, accs)

                    @pl.when(bag + 2 < nb)
                    def _():
                        gather(bag + 2, hb, b).start()

                for d in range(nd):
                    pooled_v[bag, pl.ds(d * L, L)] = accs[d]
            return carry

        lax.fori_loop(0, nb // 2, body, 0)
        pltpu.sync_copy(pooled_v, out_hbm.at[pl.ds(base, nb)])

    return bag_kernel


def _mlp_body(p_ref, w1_ref, b1_ref, w2_ref, b2_ref, o_ref):
    h = lax.dot_general(p_ref[...], w1_ref[...], (((1,), (1,)), ((), ())),
                        preferred_element_type=jnp.float32)
    h = jnp.maximum(h + b1_ref[...], 0.0)
    o_ref[...] = lax.dot_general(h, w2_ref[...], (((1,), (1,)), ((), ())),
                                 preferred_element_type=jnp.float32) + b2_ref[...]


def kernel(x, table, W1, b1, W2, b2):
    B, H = x.shape
    D = table.shape[1]
    x3 = x.reshape(B, 2, H // 2).astype(jnp.int32)
    pooled = _make_bag_kernel(B, H, D)(x3, table)
    out = pl.pallas_call(
        _mlp_body,
        out_shape=jax.ShapeDtypeStruct((B, W2.shape[0]), jnp.float32),
    )(pooled, W1, b1.reshape(1, -1), W2, b2.reshape(1, -1))
    return out
